# baseline (device time: 25591 ns/iter reference)
import jax
import jax.numpy as jnp
from jax import lax
from jax.experimental import pallas as pl
from jax.experimental.pallas import tpu as pltpu

N_DEV = 8

DIMS = ((1, 3, 4), (3, 4, 1), (4, 1, 3))

M_PARTS = (176, 176, 160)
M_OFFS = (0, 176, 352)


def _gray(dims):
    a, b, c = dims
    return (0, a, b, b ^ a, c, c ^ a, c ^ b, c ^ b ^ a)


def kernel(x):
    m_per, n = x.shape
    assert m_per == sum(M_PARTS)

    def body(x_ref, out_ref, gather_ref, send_sems, recv_sems, copy_sems):
        my = lax.axis_index("i")

        barrier_sem = pltpu.get_barrier_semaphore()
        for mask in (1, 3, 4):
            pl.semaphore_signal(
                barrier_sem, inc=1,
                device_id=(my ^ mask,), device_id_type=pl.DeviceIdType.MESH,
            )
        for j in range(3):
            gather_ref[pl.ds(my * m_per + M_OFFS[j], M_PARTS[j]), :] = x_ref[
                M_OFFS[j]:M_OFFS[j] + M_PARTS[j], :
            ].astype(jnp.bfloat16)
        pl.semaphore_wait(barrier_sem, 3)

        def vblock(j, origin):
            return gather_ref.at[pl.ds(origin * m_per + M_OFFS[j], M_PARTS[j])]

        rd = [[None] * N_DEV for _ in range(3)]
        wb = [[None] * N_DEV for _ in range(3)]

        def start(j, s, k):
            G = _gray(DIMS[j])
            origin = my ^ G[s - 2 ** k]
            rd[j][s] = pltpu.make_async_remote_copy(
                src_ref=vblock(j, origin),
                dst_ref=vblock(j, origin),
                send_sem=send_sems.at[j, s],
                recv_sem=recv_sems.at[j, s],
                device_id=(my ^ DIMS[j][k],),
                device_id_type=pl.DeviceIdType.MESH,
            )
            rd[j][s].start()

        def writeback(j, s):
            origin = my ^ _gray(DIMS[j])[s]
            wb[j][s] = pltpu.make_async_copy(
                vblock(j, origin),
                out_ref.at[pl.ds(origin * m_per + M_OFFS[j], M_PARTS[j])],
                copy_sems.at[j, s],
            )
            wb[j][s].start()

        for j in range(3):
            start(j, 1, 0)
        for j in range(3):
            start(j, 2, 1)
        for j in range(3):
            start(j, 4, 2)
        for j in range(3):
            writeback(j, 0)
        for j in range(3):
            rd[j][1].wait_recv()
            start(j, 3, 1)
            start(j, 5, 2)
            writeback(j, 1)
        for j in range(3):
            rd[j][2].wait_recv()
            rd[j][3].wait_recv()
            start(j, 6, 2)
            start(j, 7, 2)
            writeback(j, 2)
            writeback(j, 3)
        for j in range(3):
            for s in range(4, 8):
                rd[j][s].wait_recv()
                writeback(j, s)

        for j in range(3):
            for s in range(N_DEV):
                wb[j][s].wait()
        for j in range(3):
            for s in range(1, N_DEV):
                rd[j][s].wait_send()

    return pl.pallas_call(
        body,
        out_shape=jax.ShapeDtypeStruct((N_DEV * m_per, n), jnp.bfloat16),
        in_specs=[pl.BlockSpec(memory_space=pltpu.VMEM)],
        out_specs=pl.BlockSpec(memory_space=pltpu.MemorySpace.HBM),
        scratch_shapes=[
            pltpu.VMEM((N_DEV * m_per, n), jnp.bfloat16),
            pltpu.SemaphoreType.DMA((3, N_DEV)),
            pltpu.SemaphoreType.DMA((3, N_DEV)),
            pltpu.SemaphoreType.DMA((3, N_DEV)),
        ],
        compiler_params=pltpu.CompilerParams(collective_id=0),
    )(x)


# device time: 24191 ns/iter; 1.0579x vs baseline; 1.0579x over previous
import jax
import jax.numpy as jnp
from jax import lax
from jax.experimental import pallas as pl
from jax.experimental.pallas import tpu as pltpu

N_DEV = 8

DIMS = ((1, 3, 4), (3, 4, 1), (4, 1, 3))

M_PARTS = (176, 176, 160)
M_OFFS = (0, 176, 352)

TRANSFERS = (
    (0,     1,     1,   0, 0),
    (0,     1,     2,   1, 0),
    (1,     2,     3,   1, 1),
    (0,     1,     4,   2, 0),
    (1,     2,     5,   2, 1),
    (2,     4,     6,   2, 2),
)


def _gray(dims):
    a, b, c = dims
    return (0, a, b, b ^ a, c, c ^ a, c ^ b, c ^ b ^ a)


def kernel(x):
    m_per, n = x.shape
    assert m_per == sum(M_PARTS)

    def body(x_ref, out_ref, comm0, comm1, comm2, send_sems, recv_sems):
        comms = (comm0, comm1, comm2)
        my = lax.axis_index("i")

        barrier_sem = pltpu.get_barrier_semaphore()
        for mask in (1, 3, 4):
            pl.semaphore_signal(
                barrier_sem, inc=1,
                device_id=(my ^ mask,), device_id_type=pl.DeviceIdType.MESH,
            )
        for j in range(3):
            comms[j][0] = x_ref[
                M_OFFS[j]:M_OFFS[j] + M_PARTS[j], :
            ].astype(jnp.bfloat16)
        pl.semaphore_wait(barrier_sem, 3)

        rd = [[None] * len(TRANSFERS) for _ in range(3)]

        def start(j, t):
            lo, hi, dst, k, _ = TRANSFERS[t]
            rd[j][t] = pltpu.make_async_remote_copy(
                src_ref=comms[j].at[pl.ds(lo, hi - lo)],
                dst_ref=comms[j].at[pl.ds(dst, hi - lo)],
                send_sem=send_sems.at[j, t],
                recv_sem=recv_sems.at[j, t],
                device_id=(my ^ DIMS[j][k],),
                device_id_type=pl.DeviceIdType.MESH,
            )
            rd[j][t].start()

        def store(j, r):
            origin = my ^ _gray(DIMS[j])[r]
            out_ref[pl.ds(origin * m_per + M_OFFS[j], M_PARTS[j]), :] = (
                comms[j][r]
            )

        for j in range(3):
            start(j, 0)
        for j in range(3):
            start(j, 1)
            start(j, 3)
        for j in range(3):
            store(j, 0)
        for j in range(3):
            rd[j][0].wait_recv()
            start(j, 2)
            start(j, 4)
        for j in range(3):
            store(j, 1)
        for j in range(3):
            rd[j][1].wait_recv()
            rd[j][2].wait_recv()
            start(j, 5)
        for j in range(3):
            store(j, 2)
            store(j, 3)
        for j in range(3):
            rd[j][3].wait_recv()
            store(j, 4)
        for j in range(3):
            rd[j][4].wait_recv()
            store(j, 5)
        for j in range(3):
            rd[j][5].wait_recv()
            store(j, 6)
            store(j, 7)

        for j in range(3):
            for t in range(len(TRANSFERS)):
                rd[j][t].wait_send()

    return pl.pallas_call(
        body,
        out_shape=jax.ShapeDtypeStruct((N_DEV * m_per, n), jnp.bfloat16),
        in_specs=[pl.BlockSpec(memory_space=pltpu.VMEM)],
        out_specs=pl.BlockSpec(memory_space=pltpu.VMEM),
        scratch_shapes=[
            pltpu.VMEM((N_DEV, M_PARTS[0], n), jnp.bfloat16),
            pltpu.VMEM((N_DEV, M_PARTS[1], n), jnp.bfloat16),
            pltpu.VMEM((N_DEV, M_PARTS[2], n), jnp.bfloat16),
            pltpu.SemaphoreType.DMA((3, len(TRANSFERS))),
            pltpu.SemaphoreType.DMA((3, len(TRANSFERS))),
        ],
        compiler_params=pltpu.CompilerParams(collective_id=0),
    )(x)


# device time: 22300 ns/iter; 1.1476x vs baseline; 1.0848x over previous
import jax
import jax.numpy as jnp
from jax import lax
from jax.experimental import pallas as pl
from jax.experimental.pallas import tpu as pltpu

N_DEV = 8

DIMS = ((1, 3, 4), (3, 4, 1), (4, 1, 3))

M_PARTS = (176, 176, 160)
M_OFFS = (0, 176, 352)

TRANSFERS = (
    (0,     1,     1,   0, 0),
    (0,     1,     2,   1, 0),
    (1,     2,     3,   1, 1),
    (0,     1,     4,   2, 0),
    (1,     2,     5,   2, 1),
    (2,     3,     6,   2, 2),
    (3,     4,     7,   2, 3),
)


def _gray(dims):
    a, b, c = dims
    return (0, a, b, b ^ a, c, c ^ a, c ^ b, c ^ b ^ a)


def kernel(x):
    m_per, n = x.shape
    assert m_per == sum(M_PARTS)

    def body(x_ref, out_ref, comm0, comm1, comm2, send_sems, recv_sems):
        comms = (comm0, comm1, comm2)
        my = lax.axis_index("i")

        barrier_sem = pltpu.get_barrier_semaphore()
        for mask in (1, 3, 4):
            pl.semaphore_signal(
                barrier_sem, inc=1,
                device_id=(my ^ mask,), device_id_type=pl.DeviceIdType.MESH,
            )
        for j in range(3):
            comms[j][0] = x_ref[
                M_OFFS[j]:M_OFFS[j] + M_PARTS[j], :
            ].astype(jnp.bfloat16)
        pl.semaphore_wait(barrier_sem, 3)

        rd = [[None] * len(TRANSFERS) for _ in range(3)]

        def start(j, t):
            lo, hi, dst, k, _ = TRANSFERS[t]
            rd[j][t] = pltpu.make_async_remote_copy(
                src_ref=comms[j].at[pl.ds(lo, hi - lo)],
                dst_ref=comms[j].at[pl.ds(dst, hi - lo)],
                send_sem=send_sems.at[j, t],
                recv_sem=recv_sems.at[j, t],
                device_id=(my ^ DIMS[j][k],),
                device_id_type=pl.DeviceIdType.MESH,
            )
            rd[j][t].start()

        def store(j, r):
            origin = my ^ _gray(DIMS[j])[r]
            out_ref[pl.ds(origin * m_per + M_OFFS[j], M_PARTS[j]), :] = (
                comms[j][r]
            )

        for j in range(3):
            start(j, 0)
        for j in range(3):
            start(j, 1)
            start(j, 3)
        for j in range(3):
            store(j, 0)
        for j in range(3):
            rd[j][0].wait_recv()
            start(j, 2)
            start(j, 4)
        for j in range(3):
            store(j, 1)
        for j in range(3):
            rd[j][1].wait_recv()
            start(j, 5)
        for j in range(3):
            store(j, 2)
        for j in range(3):
            rd[j][2].wait_recv()
            start(j, 6)
        for j in range(3):
            store(j, 3)
        for j in range(3):
            rd[j][3].wait_recv()
            store(j, 4)
        for j in range(3):
            rd[j][4].wait_recv()
            store(j, 5)
        for j in range(3):
            rd[j][5].wait_recv()
            store(j, 6)
        for j in range(3):
            rd[j][6].wait_recv()
            store(j, 7)

        for j in range(3):
            for t in range(len(TRANSFERS)):
                rd[j][t].wait_send()

    return pl.pallas_call(
        body,
        out_shape=jax.ShapeDtypeStruct((N_DEV * m_per, n), jnp.bfloat16),
        in_specs=[pl.BlockSpec(memory_space=pltpu.VMEM)],
        out_specs=pl.BlockSpec(memory_space=pltpu.VMEM),
        scratch_shapes=[
            pltpu.VMEM((N_DEV, M_PARTS[0], n), jnp.bfloat16),
            pltpu.VMEM((N_DEV, M_PARTS[1], n), jnp.bfloat16),
            pltpu.VMEM((N_DEV, M_PARTS[2], n), jnp.bfloat16),
            pltpu.SemaphoreType.DMA((3, len(TRANSFERS))),
            pltpu.SemaphoreType.DMA((3, len(TRANSFERS))),
        ],
        compiler_params=pltpu.CompilerParams(collective_id=0),
    )(x)
